# Initial kernel scaffold; baseline (speedup 1.0000x reference)
import jax
import jax.numpy as jnp
from jax.experimental import pallas as pl


def kernel(x):
    raise NotImplementedError("write your pallas kernel here")


# speedup vs baseline: 13.4182x; 13.4182x over previous
import jax
import jax.numpy as jnp
from jax.experimental import pallas as pl
from jax.experimental.pallas import tpu as pltpu

_K = 64
_ROWS = 64
_COLS = 32768
_ROW_BLOCK = 8


def _tc_body(x_ref, o_ref):
    x = x_ref[...]
    b = jax.lax.bitcast_convert_type(x, jnp.int32)
    key = b ^ (jnp.right_shift(b, 31) & jnp.int32(0x7FFFFFFF))

    minint = jnp.int32(-(2**31))
    cnt0 = jnp.sum((key >= 0).astype(jnp.int32), axis=1, keepdims=True)
    prefix = jnp.where(cnt0 >= _K, jnp.int32(0), minint)

    def body(i, prefix):
        bit = jnp.int32(30) - i
        t = prefix | (jnp.int32(1) << bit)
        cnt = jnp.sum((key >= t).astype(jnp.int32), axis=1, keepdims=True)
        return jnp.where(cnt >= _K, t, prefix)

    prefix = jax.lax.fori_loop(0, 31, body, prefix)
    tbits = prefix ^ (jnp.right_shift(prefix, 31) & jnp.int32(0x7FFFFFFF))
    thresh = jax.lax.bitcast_convert_type(tbits, jnp.float32)
    o_ref[...] = jnp.where(x < thresh, -jnp.inf, x)


def kernel(x):
    grid = (_ROWS // _ROW_BLOCK,)
    return pl.pallas_call(
        _tc_body,
        grid=grid,
        in_specs=[pl.BlockSpec((_ROW_BLOCK, _COLS), lambda i: (i, 0))],
        out_specs=pl.BlockSpec((_ROW_BLOCK, _COLS), lambda i: (i, 0)),
        out_shape=jax.ShapeDtypeStruct((_ROWS, _COLS), jnp.float32),
    )(x)
